# baseline (speedup 1.0000x reference)
import functools

import jax
import jax.numpy as jnp
from jax import lax
from jax.experimental import pallas as pl
from jax.experimental.pallas import tpu as pltpu
from jax.experimental.pallas import tpu_sc as plsc

_NC, _NS, _L = 2, 16, 16
_NW = _NC * _NS
_C = 128
_RB = 16


@functools.lru_cache(maxsize=None)
def _make_sc_agg(N, N_pad, D, CH):
    mesh = plsc.VectorSubcoreMesh(core_axis_name="c", subcore_axis_name="s")
    rows_per_tile = N_pad // _NS
    n_full = rows_per_tile // _C

    @functools.partial(
        pl.kernel,
        out_type=jax.ShapeDtypeStruct((_NC, N_pad, D), jnp.float32),
        mesh=mesh,
        scratch_types=[
            pltpu.VMEM((_RB, _C), jnp.int32),
            pltpu.VMEM((_RB, _C), jnp.int32),
            pltpu.VMEM((_RB, _C), jnp.float32),
            pltpu.VMEM((_C, D), jnp.float32),
            pltpu.VMEM((_C, D), jnp.float32),
            pltpu.VMEM_SHARED((N_pad, D), jnp.float32),
            pltpu.SemaphoreType.DMA,
            pltpu.SemaphoreType.DMA,
            pltpu.SemaphoreType.DMA,
            pltpu.SemaphoreType.DMA,
        ],
    )
    def sc_agg(x_hbm, src_hbm, dst_hbm, w_hbm, out_hbm,
               src_r, dst_r, w_r, rows0, rows1, acc_s, g0, g1, s0, s1):
        c = lax.axis_index("c")
        s = lax.axis_index("s")
        wid = c * _NS + s
        bufs = ((rows0, g0, s0), (rows1, g1, s1))

        zeros = jnp.zeros((_L,), jnp.float32)

        def _zrow(i, _):
            for cc in range(D // _L):
                rows0[i, pl.ds(cc * _L, _L)] = zeros
            return 0

        lax.fori_loop(0, _C, _zrow, 0)
        base = pl.multiple_of(s * rows_per_tile, _C)
        for k in range(n_full):
            pltpu.sync_copy(rows0, acc_s.at[pl.ds(base + k * _C, _C)])
        plsc.subcore_barrier()

        def _refill(off):
            pltpu.sync_copy(src_hbm.at[wid, pl.ds(off, _RB)], src_r)
            pltpu.sync_copy(dst_hbm.at[wid, pl.ds(off, _RB)], dst_r)
            pltpu.sync_copy(w_hbm.at[wid, pl.ds(off, _RB)], w_r)

        _refill(0)
        pltpu.async_copy(x_hbm.at[src_r.at[0]], rows0, g0)

        def _outer(o, _):
            for b in range(2):
                rv, gs, ss = bufs[b]
                orv, ogs, oss = bufs[1 - b]
                j = o * 2 + b
                m = lax.rem(j, _RB)
                mn = lax.rem(j + 1, _RB)

                pltpu.make_async_copy(x_hbm.at[src_r.at[m]], rv, gs).wait()

                @pl.when(jnp.logical_and(j > 0, m != 0))
                def _():
                    mp = lax.rem(j + _RB - 1, _RB)
                    pltpu.make_async_copy(orv, acc_s.at[dst_r.at[mp]], oss).wait()

                @pl.when(jnp.logical_and(j + 1 < CH, mn != 0))
                def _():
                    pltpu.async_copy(x_hbm.at[src_r.at[mn]], orv, ogs)

                def _grp(g, _):
                    wv = w_r[m, pl.ds(g * _L, _L)]
                    for r in range(_L):
                        ws = wv[r]
                        i = g * _L + r
                        for cc in range(D // _L):
                            sl = pl.ds(cc * _L, _L)
                            rv[i, sl] = rv[i, sl] * ws
                    return 0

                lax.fori_loop(0, _C // _L, _grp, 0)

                pltpu.async_copy(rv, acc_s.at[dst_r.at[m]], ss, add=True)

                @pl.when(jnp.logical_and(j + 1 < CH, mn == 0))
                def _():
                    pltpu.make_async_copy(rv, acc_s.at[dst_r.at[m]], ss).wait()
                    off = pl.multiple_of(j + 1, _RB)
                    _refill(off)
                    pltpu.async_copy(x_hbm.at[src_r.at[0]], orv, ogs)

            return 0

        lax.fori_loop(0, CH // 2, _outer, 0)

        pltpu.make_async_copy(rows1, acc_s.at[dst_r.at[_RB - 1]], s1).wait()

        plsc.subcore_barrier()
        for k in range(n_full):
            sl = pl.ds(base + k * _C, _C)
            pltpu.sync_copy(acc_s.at[sl], out_hbm.at[c, sl])

    return sc_agg


def _mlp_body(p0_ref, p1_ref, W1_ref, b1_ref, W2_ref, b2_ref,
              g1_ref, be1_ref, g2_ref, be2_ref, out_ref):
    agg = p0_ref[...] + p1_ref[...]
    h = jnp.dot(agg, W1_ref[...], preferred_element_type=jnp.float32)
    h = h + b1_ref[...][None, :]
    mu = jnp.mean(h, axis=0, keepdims=True)
    var = jnp.mean((h - mu) ** 2, axis=0, keepdims=True)
    h = g1_ref[...][None, :] * (h - mu) / jnp.sqrt(var + 1e-5) + be1_ref[...][None, :]
    h = jnp.maximum(h, 0.0)
    h = jnp.dot(h, W2_ref[...], preferred_element_type=jnp.float32)
    h = h + b2_ref[...][None, :]
    mu2 = jnp.mean(h, axis=0, keepdims=True)
    var2 = jnp.mean((h - mu2) ** 2, axis=0, keepdims=True)
    h = g2_ref[...][None, :] * (h - mu2) / jnp.sqrt(var2 + 1e-5) + be2_ref[...][None, :]
    out_ref[...] = jnp.maximum(h, 0.0)


def kernel(x, edge_index, edge_weight, W1, b1, W2, b2, g1, be1, g2, be2):
    N, D = x.shape
    E = edge_weight.shape[0]
    CH = -(-E // (_NW * _C * 2 * _RB)) * 2 * _RB
    pad = _NW * _C * CH - E

    src = edge_index[0]
    dst = edge_index[1]
    w = edge_weight
    if pad:
        src = jnp.concatenate([src, jnp.zeros((pad,), src.dtype)])
        dst = jnp.concatenate([dst, jnp.zeros((pad,), dst.dtype)])
        w = jnp.concatenate([w, jnp.zeros((pad,), w.dtype)])
    src = src.reshape(_NW, CH, _C)
    dst = dst.reshape(_NW, CH, _C)
    w = w.reshape(_NW, CH, _C)

    rpt = -(-N // (_NS * _C)) * _C
    N_pad = rpt * _NS
    partials = _make_sc_agg(N, N_pad, D, CH)(x, src, dst, w)

    return pl.pallas_call(
        _mlp_body,
        out_shape=jax.ShapeDtypeStruct((N, D), jnp.float32),
    )(partials[0, :N], partials[1, :N], W1, b1, W2, b2, g1, be1, g2, be2)
